# Initial kernel scaffold; baseline (speedup 1.0000x reference)
import functools

import jax
import jax.numpy as jnp
from jax import lax
from jax.experimental import pallas as pl
from jax.experimental.pallas import tpu as pltpu
from jax.experimental.pallas import tpu_sc as plsc

VOCAB = 1000000
D = 64
BATCH = 16384
HIST = 200

NC = 2
NS = 16
NW = NC * NS

BAGS_PER_W = BATCH // NW
CHUNK = 4
ROWS = CHUNK * HIST
TR = 80
NTR = ROWS // TR
NCHUNK = BAGS_PER_W // CHUNK
IDX_ROWS = ROWS // TR


def _emb_bag_body(ids80, table, out, idx_v, rows_v, out_v, sem0, sem1):
    wid = lax.axis_index("s") * NC + lax.axis_index("c")
    w_row0 = wid * (BAGS_PER_W * HIST // TR)
    w_bag0 = wid * BAGS_PER_W

    sems = (sem0, sem1)

    def stage(g, slot, sem):
        pltpu.sync_copy(ids80.at[pl.ds(w_row0 + g * IDX_ROWS, IDX_ROWS)],
                        idx_v.at[slot])
        for k in range(NTR):
            pltpu.async_copy(table.at[idx_v.at[slot, k]],
                             rows_v.at[slot, pl.ds(k * TR, TR)], sem)

    def drain(slot, sem):
        for k in range(NTR):
            pltpu.make_async_copy(table.at[idx_v.at[slot, k]],
                                  rows_v.at[slot, pl.ds(k * TR, TR)],
                                  sem).wait()

    inv = jnp.float32(1.0 / HIST)

    def compute(g, slot):
        for i in range(CHUNK):
            base = i * HIST

            def body(j, accs):
                r = base + j
                return tuple(
                    accs[k] + rows_v[slot, r, pl.ds(k * 16, 16)]
                    for k in range(4))

            z = jnp.zeros((16,), jnp.float32)
            accs = lax.fori_loop(0, HIST, body, (z, z, z, z))
            for k in range(4):
                out_v[i, pl.ds(k * 16, 16)] = accs[k] * inv
        pltpu.sync_copy(out_v, out.at[pl.ds(w_bag0 + g * CHUNK, CHUNK)])

    stage(0, 0, sems[0])

    def pair(p, carry):
        g0 = 2 * p
        stage(g0 + 1, 1, sems[1])
        drain(0, sems[0])
        compute(g0, 0)

        @pl.when(g0 + 2 < NCHUNK)
        def _():
            stage(g0 + 2, 0, sems[0])

        drain(1, sems[1])
        compute(g0 + 1, 1)
        return carry

    lax.fori_loop(0, NCHUNK // 2, pair, 0)


@jax.jit
def _emb_bag(ids80, table):
    mesh = plsc.VectorSubcoreMesh(core_axis_name="c", subcore_axis_name="s")
    return pl.kernel(
        _emb_bag_body,
        out_type=jax.ShapeDtypeStruct((BATCH, D), jnp.float32),
        mesh=mesh,
        scratch_types=[
            pltpu.VMEM((2, IDX_ROWS, TR), jnp.int32),
            pltpu.VMEM((2, ROWS, D), jnp.float32),
            pltpu.VMEM((CHUNK, D), jnp.float32),
            pltpu.SemaphoreType.DMA,
            pltpu.SemaphoreType.DMA,
        ],
    )(ids80, table)


def _mlp_body(x_ref, w1_ref, b1_ref, w2_ref, b2_ref, wc_ref, bc_ref, o_ref):
    x = x_ref[...]
    h = jnp.maximum(
        jnp.dot(x, w1_ref[...], preferred_element_type=jnp.float32)
        + b1_ref[...], 0.0)
    h = jnp.maximum(
        jnp.dot(h, w2_ref[...], preferred_element_type=jnp.float32)
        + b2_ref[...], 0.0)
    o_ref[...] = (jnp.dot(h, wc_ref[...], preferred_element_type=jnp.float32)
                  + bc_ref[...])


@jax.jit
def _mlp(x, W1, b1, W2, b2, Wc, bc):
    BM = 2048
    nb = BATCH // BM
    rep = lambda i: (0, 0)
    return pl.pallas_call(
        _mlp_body,
        grid=(nb,),
        in_specs=[
            pl.BlockSpec((BM, D), lambda i: (i, 0)),
            pl.BlockSpec(W1.shape, rep),
            pl.BlockSpec(b1.shape, rep),
            pl.BlockSpec(W2.shape, rep),
            pl.BlockSpec(b2.shape, rep),
            pl.BlockSpec(Wc.shape, rep),
            pl.BlockSpec(bc.shape, rep),
        ],
        out_specs=pl.BlockSpec((BM, 2), lambda i: (i, 0)),
        out_shape=jax.ShapeDtypeStruct((BATCH, 2), jnp.float32),
    )(x, W1, b1, W2, b2, Wc, bc)


def kernel(input_ids, emb_table, W1, b1, W2, b2, Wc, bc):
    ids80 = input_ids.reshape(BATCH * HIST // TR, TR)
    pooled = _emb_bag(ids80, emb_table)
    return _mlp(pooled, W1, b1.reshape(1, -1), W2, b2.reshape(1, -1),
                Wc, bc.reshape(1, -1))


# speedup vs baseline: 2.9013x; 2.9013x over previous
import functools

import jax
import jax.numpy as jnp
from jax import lax
from jax.experimental import pallas as pl
from jax.experimental.pallas import tpu as pltpu
from jax.experimental.pallas import tpu_sc as plsc

VOCAB = 1000000
D = 64
BATCH = 16384
HIST = 200

NC = 2
NS = 16
NW = NC * NS

BAGS_PER_W = BATCH // NW
CHUNK = 4
ROWS = CHUNK * HIST
TR = 80
NTR = ROWS // TR
NCHUNK = BAGS_PER_W // CHUNK


def _emb_bag_body(ids_flat, table, out, idx_v, rows_v, out_v, sem0, sem1):
    wid = lax.axis_index("s") * NC + lax.axis_index("c")
    w_id0 = wid * BAGS_PER_W * HIST
    w_bag0 = wid * BAGS_PER_W

    sems = (sem0, sem1)

    def stage(g, slot, sem):
        pltpu.sync_copy(ids_flat.at[pl.ds(w_id0 + g * ROWS, ROWS)],
                        idx_v.at[pl.ds(slot * ROWS, ROWS)])
        for k in range(NTR):
            pltpu.async_copy(
                table.at[idx_v.at[pl.ds(slot * ROWS + k * TR, TR)]],
                rows_v.at[slot, pl.ds(k * TR, TR)], sem)

    def drain(slot, sem):
        for k in range(NTR):
            pltpu.make_async_copy(
                table.at[idx_v.at[pl.ds(slot * ROWS + k * TR, TR)]],
                rows_v.at[slot, pl.ds(k * TR, TR)],
                sem).wait()

    inv = jnp.float32(1.0 / HIST)

    def compute(g, slot):
        for i in range(CHUNK):
            base = i * HIST

            def body(j, accs):
                r = base + j
                return tuple(
                    accs[k] + rows_v[slot, r, pl.ds(k * 16, 16)]
                    for k in range(4))

            z = jnp.zeros((16,), jnp.float32)
            accs = lax.fori_loop(0, HIST, body, (z, z, z, z))
            for k in range(4):
                out_v[i, pl.ds(k * 16, 16)] = accs[k] * inv
        pltpu.sync_copy(out_v, out.at[pl.ds(w_bag0 + g * CHUNK, CHUNK)])

    stage(0, 0, sems[0])

    def pair(p, carry):
        g0 = 2 * p
        stage(g0 + 1, 1, sems[1])
        drain(0, sems[0])
        compute(g0, 0)

        @pl.when(g0 + 2 < NCHUNK)
        def _():
            stage(g0 + 2, 0, sems[0])

        drain(1, sems[1])
        compute(g0 + 1, 1)
        return carry

    lax.fori_loop(0, NCHUNK // 2, pair, 0)


@jax.jit
def _emb_bag(ids80, table):
    mesh = plsc.VectorSubcoreMesh(core_axis_name="c", subcore_axis_name="s")
    return pl.kernel(
        _emb_bag_body,
        out_type=jax.ShapeDtypeStruct((BATCH, D), jnp.float32),
        mesh=mesh,
        scratch_types=[
            pltpu.VMEM((2 * ROWS,), jnp.int32),
            pltpu.VMEM((2, ROWS, D), jnp.float32),
            pltpu.VMEM((CHUNK, D), jnp.float32),
            pltpu.SemaphoreType.DMA,
            pltpu.SemaphoreType.DMA,
        ],
        compiler_params=pltpu.CompilerParams(use_tc_tiling_on_sc=False),
    )(ids80, table)


def _mlp_body(x_ref, w1_ref, b1_ref, w2_ref, b2_ref, wc_ref, bc_ref, o_ref):
    x = x_ref[...]
    h = jnp.maximum(
        jnp.dot(x, w1_ref[...], preferred_element_type=jnp.float32)
        + b1_ref[...], 0.0)
    h = jnp.maximum(
        jnp.dot(h, w2_ref[...], preferred_element_type=jnp.float32)
        + b2_ref[...], 0.0)
    o_ref[...] = (jnp.dot(h, wc_ref[...], preferred_element_type=jnp.float32)
                  + bc_ref[...])


@jax.jit
def _mlp(x, W1, b1, W2, b2, Wc, bc):
    BM = 2048
    nb = BATCH // BM
    rep = lambda i: (0, 0)
    return pl.pallas_call(
        _mlp_body,
        grid=(nb,),
        in_specs=[
            pl.BlockSpec((BM, D), lambda i: (i, 0)),
            pl.BlockSpec(W1.shape, rep),
            pl.BlockSpec(b1.shape, rep),
            pl.BlockSpec(W2.shape, rep),
            pl.BlockSpec(b2.shape, rep),
            pl.BlockSpec(Wc.shape, rep),
            pl.BlockSpec(bc.shape, rep),
        ],
        out_specs=pl.BlockSpec((BM, 2), lambda i: (i, 0)),
        out_shape=jax.ShapeDtypeStruct((BATCH, 2), jnp.float32),
    )(x, W1, b1, W2, b2, Wc, bc)


def kernel(input_ids, emb_table, W1, b1, W2, b2, Wc, bc):
    ids_flat = input_ids.reshape(BATCH * HIST)
    pooled = _emb_bag(ids_flat, emb_table)
    return _mlp(pooled, W1, b1.reshape(1, -1), W2, b2.reshape(1, -1),
                Wc, bc.reshape(1, -1))
